# baseline (speedup 1.0000x reference)
import functools

import jax
import jax.numpy as jnp
from jax import lax
from jax.experimental import pallas as pl
from jax.experimental.pallas import tpu as pltpu
from jax.experimental.pallas import tpu_sc as plsc

B = 8
N = 4096
D = 64
S = 512
K = 32
RADIUS = 0.5
M = B * S * K
MF = float(M)
BIG = 1e30


def _fps_body(xyz_ref, out_ref):
    x0 = xyz_ref[:, 0, :]
    x1 = xyz_ref[:, 1, :]
    x2 = xyz_ref[:, 2, :]
    iota_n = lax.broadcasted_iota(jnp.int32, (B, N), 1)
    lane_s = lax.broadcasted_iota(jnp.int32, (B, S), 1)

    def step(t, carry):
        dist, far, o0, o1, o2 = carry
        sel = iota_n == far
        c0 = jnp.sum(jnp.where(sel, x0, 0.0), axis=1, keepdims=True)
        c1 = jnp.sum(jnp.where(sel, x1, 0.0), axis=1, keepdims=True)
        c2 = jnp.sum(jnp.where(sel, x2, 0.0), axis=1, keepdims=True)
        rec = lane_s == t
        o0 = jnp.where(rec, c0, o0)
        o1 = jnp.where(rec, c1, o1)
        o2 = jnp.where(rec, c2, o2)
        d = (x0 - c0) ** 2 + (x1 - c1) ** 2 + (x2 - c2) ** 2
        dist = jnp.minimum(dist, d)
        m = jnp.max(dist, axis=1, keepdims=True)
        far = jnp.min(jnp.where(dist == m, iota_n, N), axis=1, keepdims=True)
        return dist, far, o0, o1, o2

    init = (jnp.full((B, N), 1e10, jnp.float32),
            jnp.zeros((B, 1), jnp.int32),
            jnp.zeros((B, S), jnp.float32),
            jnp.zeros((B, S), jnp.float32),
            jnp.zeros((B, S), jnp.float32))
    _, _, o0, o1, o2 = lax.fori_loop(0, S, step, init)
    out_ref[:, 0, :] = o0
    out_ref[:, 1, :] = o1
    out_ref[:, 2, :] = o2


def _fps(xyz):
    return pl.pallas_call(
        _fps_body,
        out_shape=jax.ShapeDtypeStruct((B, 3, S), jnp.float32),
    )(xyz)


def _bq_body(xyz_ref, pts_ref, nx_ref, w0x_ref, w0p_ref, b0_ref,
             idx_ref, q_ref, p_ref):
    b = pl.program_id(0)
    xyz_b = xyz_ref[0]
    nx = nx_ref[0]

    s2 = jnp.sum(nx ** 2, axis=1, keepdims=True)
    x2 = jnp.sum(xyz_b ** 2, axis=0, keepdims=True)
    dots = lax.dot_general(nx, xyz_b, (((1,), (0,)), ((), ())),
                           preferred_element_type=jnp.float32)
    sq = (s2 + x2) - 2.0 * dots
    sqrd = jnp.sqrt(jnp.maximum(sq, 0.0))

    iota_n = lax.broadcasted_iota(jnp.int32, (S, N), 1)
    lane_k = lax.broadcasted_iota(jnp.int32, (S, K), 1)
    base = b * N
    BIG_I = jnp.int32(0x7F000000)
    bits = lax.bitcast_convert_type(sq, jnp.int32)
    packed = jnp.bitwise_or(jnp.bitwise_and(bits, jnp.int32(-4096)), iota_n)
    key0 = jnp.where(sqrd < RADIUS * RADIUS, packed, BIG_I)

    def step(t, carry):
        key, out = carry
        rowmin = jnp.min(key, axis=1, keepdims=True)
        valid = rowmin < BIG_I
        pick = jnp.where(valid,
                         jnp.bitwise_and(rowmin, jnp.int32(4095)) + base,
                         base)
        out = jnp.where(lane_k == t, pick, out)
        key = jnp.where(key == rowmin, BIG_I, key)
        return key, out

    _, out_idx = lax.fori_loop(0, K, step,
                               (key0, jnp.zeros((S, K), jnp.int32)))
    idx_ref[0] = out_idx

    q_ref[0] = lax.dot_general(nx, w0x_ref[...], (((1,), (1,)), ((), ())),
                               preferred_element_type=jnp.float32)

    p = lax.dot_general(xyz_b, w0x_ref[...], (((0,), (1,)), ((), ())),
                        preferred_element_type=jnp.float32)
    p = p + lax.dot_general(pts_ref[0], w0p_ref[...], (((0,), (1,)), ((), ())),
                            preferred_element_type=jnp.float32)
    p_ref[0] = p + b0_ref[...]


def _bq(xyz, points, new_xyz_t, w0x, w0p, b0row, nb):
    return pl.pallas_call(
        _bq_body,
        grid=(nb,),
        in_specs=[
            pl.BlockSpec((1, 3, N), lambda b: (b, 0, 0)),
            pl.BlockSpec((1, D, N), lambda b: (b, 0, 0)),
            pl.BlockSpec((1, S, 3), lambda b: (b, 0, 0)),
            pl.BlockSpec((128, 3), lambda b: (0, 0)),
            pl.BlockSpec((128, D), lambda b: (0, 0)),
            pl.BlockSpec((1, 128), lambda b: (0, 0)),
        ],
        out_specs=[
            pl.BlockSpec((1, S, K), lambda b: (b, 0, 0)),
            pl.BlockSpec((1, S, 128), lambda b: (b, 0, 0)),
            pl.BlockSpec((1, N, 128), lambda b: (b, 0, 0)),
        ],
        out_shape=[
            jax.ShapeDtypeStruct((nb, S, K), jnp.int32),
            jax.ShapeDtypeStruct((nb, S, 128), jnp.float32),
            jax.ShapeDtypeStruct((nb, N, 128), jnp.float32),
        ],
    )(xyz, points, new_xyz_t, w0x, w0p, b0row)


_CHUNK = 128
_PER_W = M // 32


_NB = 4


def _sc_gather_call(p_flat, gidx2d, n_rows):
    per_w = n_rows // 32
    nch = per_w // _CHUNK
    mesh = plsc.VectorSubcoreMesh(core_axis_name="c", subcore_axis_name="s")

    @functools.partial(
        pl.kernel,
        out_type=jax.ShapeDtypeStruct((n_rows, 128), jnp.float32),
        scratch_types=[
            pltpu.VMEM((nch, _CHUNK), jnp.int32),
            pltpu.VMEM((_CHUNK, 128), jnp.float32),
            pltpu.VMEM((_CHUNK, 128), jnp.float32),
            pltpu.VMEM((_CHUNK, 128), jnp.float32),
            pltpu.VMEM((_CHUNK, 128), jnp.float32),
            pltpu.SemaphoreType.DMA,
            pltpu.SemaphoreType.DMA,
        ],
        mesh=mesh,
    )
    def gather_kernel(p_hbm, gidx_hbm, out_hbm,
                      idx_v, r0, r1, r2, r3, gsem, osem):
        rows = (r0, r1, r2, r3)
        wid = lax.axis_index("s") * 2 + lax.axis_index("c")
        w_base = wid * per_w
        pltpu.sync_copy(gidx_hbm.at[pl.ds(wid * nch, nch)], idx_v)

        def group(g, _):
            @pl.when(g > 0)
            def _():
                for b in range(_NB):
                    jp = (g - 1) * _NB + b
                    pltpu.make_async_copy(
                        rows[b],
                        out_hbm.at[pl.ds(w_base + jp * _CHUNK, _CHUNK)],
                        osem).wait()
            for b in range(_NB):
                j = g * _NB + b
                pltpu.async_copy(p_hbm.at[idx_v.at[j]], rows[b], gsem)
            for b in range(_NB):
                j = g * _NB + b
                pltpu.make_async_copy(p_hbm.at[idx_v.at[j]], rows[b],
                                      gsem).wait()
                pltpu.async_copy(
                    rows[b], out_hbm.at[pl.ds(w_base + j * _CHUNK, _CHUNK)],
                    osem)
            return 0

        lax.fori_loop(0, nch // _NB, group, 0)
        for b in range(_NB):
            jp = nch - _NB + b
            pltpu.make_async_copy(
                rows[b], out_hbm.at[pl.ds(w_base + jp * _CHUNK, _CHUNK)],
                osem).wait()

    return gather_kernel(p_flat, gidx2d)


def _sc_gather_rows(p_flat, gidx, n_rows):
    return _sc_gather_call(p_flat, gidx, n_rows)


def _stats0_body(g_ref, q_ref, acc_ref):
    @pl.when(pl.program_id(0) == 0)
    def _():
        acc_ref[...] = jnp.zeros_like(acc_ref)

    q = q_ref[0]
    s = jnp.zeros((1, 128), jnp.float32)
    s2 = jnp.zeros((1, 128), jnp.float32)
    for j in range(8):
        z = g_ref[j * S:(j + 1) * S, :] - q
        s = s + jnp.sum(z, axis=0, keepdims=True)
        s2 = s2 + jnp.sum(z * z, axis=0, keepdims=True)
    acc_ref[0:1, :] += s
    acc_ref[1:2, :] += s2


def _stats0(g, q):
    return pl.pallas_call(
        _stats0_body,
        grid=(32,),
        in_specs=[
            pl.BlockSpec((4096, 128), lambda i: (i, 0)),
            pl.BlockSpec((1, S, 128), lambda i: (i // 4, 0, 0)),
        ],
        out_specs=pl.BlockSpec((2, 128), lambda i: (0, 0)),
        out_shape=jax.ShapeDtypeStruct((2, 128), jnp.float32),
    )(g, q)


def _passb_body(g_ref, q_ref, st_ref, g0_ref, be0_ref, w1_ref, b1_ref,
                y1_ref, acc_ref):
    @pl.when(pl.program_id(0) == 0)
    def _():
        acc_ref[...] = jnp.zeros_like(acc_ref)

    m = st_ref[0:1, :] / MF
    v = st_ref[1:2, :] / MF - m * m
    inv = 1.0 / jnp.sqrt(v + 1e-5)
    scale = inv * g0_ref[...]
    shift = be0_ref[...] - m * scale

    q = q_ref[0]
    parts = []
    for j in range(8):
        z = g_ref[j * S:(j + 1) * S, :] - q
        parts.append(jnp.maximum(z * scale + shift, 0.0))
    a = jnp.concatenate(parts, axis=0)
    y1 = lax.dot_general(a, w1_ref[...], (((1,), (1,)), ((), ())),
                         preferred_element_type=jnp.float32) + b1_ref[...]
    y1_ref[...] = y1
    acc_ref[0:1, :] += jnp.sum(y1, axis=0, keepdims=True)
    acc_ref[1:2, :] += jnp.sum(y1 * y1, axis=0, keepdims=True)


def _passb(g, q, st0, g0row, be0row, w1, b1row):
    return pl.pallas_call(
        _passb_body,
        grid=(32,),
        in_specs=[
            pl.BlockSpec((4096, 128), lambda i: (i, 0)),
            pl.BlockSpec((1, S, 128), lambda i: (i // 4, 0, 0)),
            pl.BlockSpec((2, 128), lambda i: (0, 0)),
            pl.BlockSpec((1, 128), lambda i: (0, 0)),
            pl.BlockSpec((1, 128), lambda i: (0, 0)),
            pl.BlockSpec((128, 128), lambda i: (0, 0)),
            pl.BlockSpec((1, 128), lambda i: (0, 0)),
        ],
        out_specs=[
            pl.BlockSpec((4096, 128), lambda i: (i, 0)),
            pl.BlockSpec((2, 128), lambda i: (0, 0)),
        ],
        out_shape=[
            jax.ShapeDtypeStruct((M, 128), jnp.float32),
            jax.ShapeDtypeStruct((2, 128), jnp.float32),
        ],
    )(g, q, st0, g0row, be0row, w1, b1row)


def _passc_body(y1_ref, st_ref, g1_ref, be1_ref, w2_ref, b2_ref,
                mx_ref, mn_ref, acc_ref):
    @pl.when(pl.program_id(0) == 0)
    def _():
        acc_ref[...] = jnp.zeros_like(acc_ref)

    m = st_ref[0:1, :] / MF
    v = st_ref[1:2, :] / MF - m * m
    inv = 1.0 / jnp.sqrt(v + 1e-5)
    scale = inv * g1_ref[...]
    shift = be1_ref[...] - m * scale

    a = jnp.maximum(y1_ref[...] * scale + shift, 0.0)
    y2 = lax.dot_general(a, w2_ref[...], (((1,), (1,)), ((), ())),
                         preferred_element_type=jnp.float32) + b2_ref[...]
    acc_ref[0:1, :] += jnp.sum(y2, axis=0, keepdims=True)
    acc_ref[1:2, :] += jnp.sum(y2 * y2, axis=0, keepdims=True)
    mx = y2[0:S, :]
    mn = y2[0:S, :]
    for k in range(1, K):
        blk = y2[k * S:(k + 1) * S, :]
        mx = jnp.maximum(mx, blk)
        mn = jnp.minimum(mn, blk)
    mx_ref[...] = mx
    mn_ref[...] = mn


def _passc(y1, st1, g1row, be1row, w2, b2row):
    return pl.pallas_call(
        _passc_body,
        grid=(B,),
        in_specs=[
            pl.BlockSpec((S * K, 128), lambda b: (b, 0)),
            pl.BlockSpec((2, 128), lambda b: (0, 0)),
            pl.BlockSpec((1, 128), lambda b: (0, 0)),
            pl.BlockSpec((1, 128), lambda b: (0, 0)),
            pl.BlockSpec((256, 128), lambda b: (0, 0)),
            pl.BlockSpec((1, 256), lambda b: (0, 0)),
        ],
        out_specs=[
            pl.BlockSpec((S, 256), lambda b: (b, 0)),
            pl.BlockSpec((S, 256), lambda b: (b, 0)),
            pl.BlockSpec((2, 256), lambda b: (0, 0)),
        ],
        out_shape=[
            jax.ShapeDtypeStruct((B * S, 256), jnp.float32),
            jax.ShapeDtypeStruct((B * S, 256), jnp.float32),
            jax.ShapeDtypeStruct((2, 256), jnp.float32),
        ],
    )(y1, st1, g1row, be1row, w2, b2row)


def _passd_body(mx_ref, mn_ref, st_ref, g2_ref, be2_ref, out_ref):
    m = st_ref[0:1, :] / MF
    v = st_ref[1:2, :] / MF - m * m
    inv = 1.0 / jnp.sqrt(v + 1e-5)
    scale = inv * g2_ref[...]
    shift = be2_ref[...] - m * scale

    picked = jnp.where(scale > 0.0, mx_ref[...], mn_ref[...])
    r = jnp.maximum(picked * scale + shift, 0.0)
    eye = (lax.broadcasted_iota(jnp.int32, (S, S), 0)
           == lax.broadcasted_iota(jnp.int32, (S, S), 1)).astype(jnp.float32)
    out_ref[0] = lax.dot_general(r, eye, (((0,), (0,)), ((), ())),
                                 preferred_element_type=jnp.float32)


def _passd(mx, mn, st2, g2row, be2row):
    return pl.pallas_call(
        _passd_body,
        grid=(B,),
        in_specs=[
            pl.BlockSpec((S, 256), lambda b: (b, 0)),
            pl.BlockSpec((S, 256), lambda b: (b, 0)),
            pl.BlockSpec((2, 256), lambda b: (0, 0)),
            pl.BlockSpec((1, 256), lambda b: (0, 0)),
            pl.BlockSpec((1, 256), lambda b: (0, 0)),
        ],
        out_specs=pl.BlockSpec((1, 256, S), lambda b: (b, 0, 0)),
        out_shape=jax.ShapeDtypeStruct((B, 256, S), jnp.float32),
    )(mx, mn, st2, g2row, be2row)


def kernel(xyz, points, W0, b0, g0, be0, W1, b1, g1, be1, W2, b2, g2, be2):
    w0x = W0[:, :3]
    w0p = W0[:, 3:]
    b0row = b0.reshape(1, 128)
    new_xyz = _fps(xyz)
    new_xyz_t = jnp.transpose(new_xyz, (0, 2, 1))
    H = B // 2
    MH = M // 2
    idx_a, q_a, p_a = _bq(xyz[:H], points[:H], new_xyz_t[:H],
                          w0x, w0p, b0row, H)
    gidx_a = jnp.transpose(idx_a, (0, 2, 1)).reshape(MH // _CHUNK, _CHUNK)
    idx_b, q_b, p_b = _bq(xyz[H:], points[H:], new_xyz_t[H:],
                          w0x, w0p, b0row, H)
    gidx_b = jnp.transpose(idx_b, (0, 2, 1)).reshape(MH // _CHUNK, _CHUNK)
    g_a = _sc_gather_rows(p_a.reshape(H * N, 128), gidx_a, MH)
    g_b = _sc_gather_rows(p_b.reshape(H * N, 128), gidx_b, MH)
    g = jnp.concatenate([g_a, g_b], axis=0)
    q = jnp.concatenate([q_a, q_b], axis=0)
    st0 = _stats0(g, q)
    y1, st1 = _passb(g, q, st0, g0.reshape(1, 128), be0.reshape(1, 128),
                     W1, b1.reshape(1, 128))
    mx, mn, st2 = _passc(y1, st1, g1.reshape(1, 128), be1.reshape(1, 128),
                         W2, b2.reshape(1, 256))
    new_points = _passd(mx, mn, st2, g2.reshape(1, 256), be2.reshape(1, 256))
    return (new_xyz, new_points)


# speedup vs baseline: 1.2056x; 1.2056x over previous
import functools

import jax
import jax.numpy as jnp
from jax import lax
from jax.experimental import pallas as pl
from jax.experimental.pallas import tpu as pltpu
from jax.experimental.pallas import tpu_sc as plsc

B = 8
N = 4096
D = 64
S = 512
K = 32
RADIUS = 0.5
M = B * S * K
MF = float(M)
BIG = 1e30


def _fps_body(xyz_ref, out_ref):
    x0 = xyz_ref[:, 0, :]
    x1 = xyz_ref[:, 1, :]
    x2 = xyz_ref[:, 2, :]
    iota_n = lax.broadcasted_iota(jnp.int32, (B, N), 1)
    lane_s = lax.broadcasted_iota(jnp.int32, (B, S), 1)

    def step(t, carry):
        dist, far, o0, o1, o2, o3 = carry
        sel = iota_n == far
        c0 = jnp.sum(jnp.where(sel, x0, 0.0), axis=1, keepdims=True)
        c1 = jnp.sum(jnp.where(sel, x1, 0.0), axis=1, keepdims=True)
        c2 = jnp.sum(jnp.where(sel, x2, 0.0), axis=1, keepdims=True)
        rec = lane_s == t
        o0 = jnp.where(rec, c0, o0)
        o1 = jnp.where(rec, c1, o1)
        o2 = jnp.where(rec, c2, o2)
        o3 = jnp.where(rec, (c0 * c0 + c1 * c1) + c2 * c2, o3)
        d = (x0 - c0) ** 2 + (x1 - c1) ** 2 + (x2 - c2) ** 2
        dist = jnp.minimum(dist, d)
        m = jnp.max(dist, axis=1, keepdims=True)
        far = jnp.min(jnp.where(dist == m, iota_n, N), axis=1, keepdims=True)
        return dist, far, o0, o1, o2, o3

    init = (jnp.full((B, N), 1e10, jnp.float32),
            jnp.zeros((B, 1), jnp.int32),
            jnp.zeros((B, S), jnp.float32),
            jnp.zeros((B, S), jnp.float32),
            jnp.zeros((B, S), jnp.float32),
            jnp.zeros((B, S), jnp.float32))
    _, _, o0, o1, o2, o3 = lax.fori_loop(0, S, step, init)
    out_ref[:, 0, :] = o0
    out_ref[:, 1, :] = o1
    out_ref[:, 2, :] = o2
    out_ref[:, 3, :] = o3


def _fps(xyz):
    return pl.pallas_call(
        _fps_body,
        out_shape=jax.ShapeDtypeStruct((B, 4, S), jnp.float32),
    )(xyz)


def _bq_body(xyzt_ref, pts_ref, nx_ref, s2_ref, w0x_ref, w0p_ref, b0_ref,
             idx_ref, q_ref, p_ref):
    b = pl.program_id(0)
    xyzt = xyzt_ref[0]
    nx = nx_ref[0]
    s2 = s2_ref[0]

    x2 = jnp.sum(xyzt ** 2, axis=1, keepdims=True)
    dots = lax.dot_general(xyzt, nx, (((1,), (1,)), ((), ())),
                           preferred_element_type=jnp.float32)
    sq = (s2 + x2) - 2.0 * dots
    sqrd = jnp.sqrt(jnp.maximum(sq, 0.0))

    iota_n = lax.broadcasted_iota(jnp.int32, (N, S), 0)
    row_k = lax.broadcasted_iota(jnp.int32, (K, S), 0)
    base = b * N
    BIG_I = jnp.int32(0x7F000000)
    bits = lax.bitcast_convert_type(sq, jnp.int32)
    packed = jnp.bitwise_or(jnp.bitwise_and(bits, jnp.int32(-4096)), iota_n)
    key0 = jnp.where(sqrd < RADIUS * RADIUS, packed, BIG_I)

    def step(t, carry):
        key, out = carry
        colmin = jnp.min(key, axis=0, keepdims=True)
        valid = colmin < BIG_I
        pick = jnp.where(valid,
                         jnp.bitwise_and(colmin, jnp.int32(4095)) + base,
                         base)
        out = jnp.where(row_k == t, pick, out)
        key = jnp.where(key == colmin, BIG_I, key)
        return key, out

    _, out_idx = lax.fori_loop(0, K, step,
                               (key0, jnp.zeros((K, S), jnp.int32)))
    idx_ref[0] = out_idx

    q_ref[0] = lax.dot_general(nx, w0x_ref[...], (((1,), (1,)), ((), ())),
                               preferred_element_type=jnp.float32)

    p = lax.dot_general(xyzt, w0x_ref[...], (((1,), (1,)), ((), ())),
                        preferred_element_type=jnp.float32)
    p = p + lax.dot_general(pts_ref[0], w0p_ref[...], (((0,), (1,)), ((), ())),
                            preferred_element_type=jnp.float32)
    p_ref[0] = p + b0_ref[...]


def _bq(xyz_t, points, new_xyz_t, s2row, w0x, w0p, b0row, nb):
    return pl.pallas_call(
        _bq_body,
        grid=(nb,),
        in_specs=[
            pl.BlockSpec((1, N, 3), lambda b: (b, 0, 0)),
            pl.BlockSpec((1, D, N), lambda b: (b, 0, 0)),
            pl.BlockSpec((1, S, 3), lambda b: (b, 0, 0)),
            pl.BlockSpec((1, 1, S), lambda b: (b, 0, 0)),
            pl.BlockSpec((128, 3), lambda b: (0, 0)),
            pl.BlockSpec((128, D), lambda b: (0, 0)),
            pl.BlockSpec((1, 128), lambda b: (0, 0)),
        ],
        out_specs=[
            pl.BlockSpec((1, K, S), lambda b: (b, 0, 0)),
            pl.BlockSpec((1, S, 128), lambda b: (b, 0, 0)),
            pl.BlockSpec((1, N, 128), lambda b: (b, 0, 0)),
        ],
        out_shape=[
            jax.ShapeDtypeStruct((nb, K, S), jnp.int32),
            jax.ShapeDtypeStruct((nb, S, 128), jnp.float32),
            jax.ShapeDtypeStruct((nb, N, 128), jnp.float32),
        ],
    )(xyz_t, points, new_xyz_t, s2row, w0x, w0p, b0row)


_CHUNK = 128
_PER_W = M // 32


_NB = 4


def _sc_gather_call(p_flat, gidx2d, n_rows):
    per_w = n_rows // 32
    nch = per_w // _CHUNK
    mesh = plsc.VectorSubcoreMesh(core_axis_name="c", subcore_axis_name="s")

    @functools.partial(
        pl.kernel,
        out_type=jax.ShapeDtypeStruct((n_rows, 128), jnp.float32),
        scratch_types=[
            pltpu.VMEM((nch, _CHUNK), jnp.int32),
            pltpu.VMEM((_CHUNK, 128), jnp.float32),
            pltpu.VMEM((_CHUNK, 128), jnp.float32),
            pltpu.VMEM((_CHUNK, 128), jnp.float32),
            pltpu.VMEM((_CHUNK, 128), jnp.float32),
            pltpu.SemaphoreType.DMA,
            pltpu.SemaphoreType.DMA,
        ],
        mesh=mesh,
    )
    def gather_kernel(p_hbm, gidx_hbm, out_hbm,
                      idx_v, r0, r1, r2, r3, gsem, osem):
        rows = (r0, r1, r2, r3)
        wid = lax.axis_index("s") * 2 + lax.axis_index("c")
        w_base = wid * per_w
        pltpu.sync_copy(gidx_hbm.at[pl.ds(wid * nch, nch)], idx_v)

        def group(g, _):
            @pl.when(g > 0)
            def _():
                for b in range(_NB):
                    jp = (g - 1) * _NB + b
                    pltpu.make_async_copy(
                        rows[b],
                        out_hbm.at[pl.ds(w_base + jp * _CHUNK, _CHUNK)],
                        osem).wait()
            for b in range(_NB):
                j = g * _NB + b
                pltpu.async_copy(p_hbm.at[idx_v.at[j]], rows[b], gsem)
            for b in range(_NB):
                j = g * _NB + b
                pltpu.make_async_copy(p_hbm.at[idx_v.at[j]], rows[b],
                                      gsem).wait()
                pltpu.async_copy(
                    rows[b], out_hbm.at[pl.ds(w_base + j * _CHUNK, _CHUNK)],
                    osem)
            return 0

        lax.fori_loop(0, nch // _NB, group, 0)
        for b in range(_NB):
            jp = nch - _NB + b
            pltpu.make_async_copy(
                rows[b], out_hbm.at[pl.ds(w_base + jp * _CHUNK, _CHUNK)],
                osem).wait()

    return gather_kernel(p_flat, gidx2d)


def _sc_gather_rows(p_flat, gidx, n_rows):
    return _sc_gather_call(p_flat, gidx, n_rows)


def _stats0_body(g_ref, q_ref, acc_ref):
    @pl.when(pl.program_id(0) == 0)
    def _():
        acc_ref[...] = jnp.zeros_like(acc_ref)

    q = q_ref[0]
    s = jnp.zeros((1, 128), jnp.float32)
    s2 = jnp.zeros((1, 128), jnp.float32)
    for j in range(8):
        z = g_ref[j * S:(j + 1) * S, :] - q
        s = s + jnp.sum(z, axis=0, keepdims=True)
        s2 = s2 + jnp.sum(z * z, axis=0, keepdims=True)
    acc_ref[0:1, :] += s
    acc_ref[1:2, :] += s2


def _stats0(g, q):
    return pl.pallas_call(
        _stats0_body,
        grid=(32,),
        in_specs=[
            pl.BlockSpec((4096, 128), lambda i: (i, 0)),
            pl.BlockSpec((1, S, 128), lambda i: (i // 4, 0, 0)),
        ],
        out_specs=pl.BlockSpec((2, 128), lambda i: (0, 0)),
        out_shape=jax.ShapeDtypeStruct((2, 128), jnp.float32),
    )(g, q)


def _passb_body(g_ref, q_ref, st_ref, g0_ref, be0_ref, w1_ref, b1_ref,
                y1_ref, acc_ref):
    @pl.when(pl.program_id(0) == 0)
    def _():
        acc_ref[...] = jnp.zeros_like(acc_ref)

    m = st_ref[0:1, :] / MF
    v = st_ref[1:2, :] / MF - m * m
    inv = 1.0 / jnp.sqrt(v + 1e-5)
    scale = inv * g0_ref[...]
    shift = be0_ref[...] - m * scale

    q = q_ref[0]
    parts = []
    for j in range(8):
        z = g_ref[j * S:(j + 1) * S, :] - q
        parts.append(jnp.maximum(z * scale + shift, 0.0))
    a = jnp.concatenate(parts, axis=0)
    y1 = lax.dot_general(a, w1_ref[...], (((1,), (1,)), ((), ())),
                         preferred_element_type=jnp.float32) + b1_ref[...]
    y1_ref[...] = y1
    acc_ref[0:1, :] += jnp.sum(y1, axis=0, keepdims=True)
    acc_ref[1:2, :] += jnp.sum(y1 * y1, axis=0, keepdims=True)


def _passb(g, q, st0, g0row, be0row, w1, b1row):
    return pl.pallas_call(
        _passb_body,
        grid=(32,),
        in_specs=[
            pl.BlockSpec((4096, 128), lambda i: (i, 0)),
            pl.BlockSpec((1, S, 128), lambda i: (i // 4, 0, 0)),
            pl.BlockSpec((2, 128), lambda i: (0, 0)),
            pl.BlockSpec((1, 128), lambda i: (0, 0)),
            pl.BlockSpec((1, 128), lambda i: (0, 0)),
            pl.BlockSpec((128, 128), lambda i: (0, 0)),
            pl.BlockSpec((1, 128), lambda i: (0, 0)),
        ],
        out_specs=[
            pl.BlockSpec((4096, 128), lambda i: (i, 0)),
            pl.BlockSpec((2, 128), lambda i: (0, 0)),
        ],
        out_shape=[
            jax.ShapeDtypeStruct((M, 128), jnp.float32),
            jax.ShapeDtypeStruct((2, 128), jnp.float32),
        ],
    )(g, q, st0, g0row, be0row, w1, b1row)


def _passc_body(y1_ref, st_ref, g1_ref, be1_ref, w2_ref, b2_ref,
                mx_ref, mn_ref, acc_ref):
    @pl.when(pl.program_id(0) == 0)
    def _():
        acc_ref[...] = jnp.zeros_like(acc_ref)

    m = st_ref[0:1, :] / MF
    v = st_ref[1:2, :] / MF - m * m
    inv = 1.0 / jnp.sqrt(v + 1e-5)
    scale = inv * g1_ref[...]
    shift = be1_ref[...] - m * scale

    a = jnp.maximum(y1_ref[...] * scale + shift, 0.0)
    y2 = lax.dot_general(a, w2_ref[...], (((1,), (1,)), ((), ())),
                         preferred_element_type=jnp.float32) + b2_ref[...]
    acc_ref[0:1, :] += jnp.sum(y2, axis=0, keepdims=True)
    acc_ref[1:2, :] += jnp.sum(y2 * y2, axis=0, keepdims=True)
    mx = y2[0:S, :]
    mn = y2[0:S, :]
    for k in range(1, K):
        blk = y2[k * S:(k + 1) * S, :]
        mx = jnp.maximum(mx, blk)
        mn = jnp.minimum(mn, blk)
    mx_ref[...] = mx
    mn_ref[...] = mn


def _passc(y1, st1, g1row, be1row, w2, b2row):
    return pl.pallas_call(
        _passc_body,
        grid=(B,),
        in_specs=[
            pl.BlockSpec((S * K, 128), lambda b: (b, 0)),
            pl.BlockSpec((2, 128), lambda b: (0, 0)),
            pl.BlockSpec((1, 128), lambda b: (0, 0)),
            pl.BlockSpec((1, 128), lambda b: (0, 0)),
            pl.BlockSpec((256, 128), lambda b: (0, 0)),
            pl.BlockSpec((1, 256), lambda b: (0, 0)),
        ],
        out_specs=[
            pl.BlockSpec((S, 256), lambda b: (b, 0)),
            pl.BlockSpec((S, 256), lambda b: (b, 0)),
            pl.BlockSpec((2, 256), lambda b: (0, 0)),
        ],
        out_shape=[
            jax.ShapeDtypeStruct((B * S, 256), jnp.float32),
            jax.ShapeDtypeStruct((B * S, 256), jnp.float32),
            jax.ShapeDtypeStruct((2, 256), jnp.float32),
        ],
    )(y1, st1, g1row, be1row, w2, b2row)


def _passd_body(mx_ref, mn_ref, st_ref, g2_ref, be2_ref, out_ref):
    m = st_ref[0:1, :] / MF
    v = st_ref[1:2, :] / MF - m * m
    inv = 1.0 / jnp.sqrt(v + 1e-5)
    scale = inv * g2_ref[...]
    shift = be2_ref[...] - m * scale

    picked = jnp.where(scale > 0.0, mx_ref[...], mn_ref[...])
    r = jnp.maximum(picked * scale + shift, 0.0)
    eye = (lax.broadcasted_iota(jnp.int32, (S, S), 0)
           == lax.broadcasted_iota(jnp.int32, (S, S), 1)).astype(jnp.float32)
    out_ref[0] = lax.dot_general(r, eye, (((0,), (0,)), ((), ())),
                                 preferred_element_type=jnp.float32)


def _passd(mx, mn, st2, g2row, be2row):
    return pl.pallas_call(
        _passd_body,
        grid=(B,),
        in_specs=[
            pl.BlockSpec((S, 256), lambda b: (b, 0)),
            pl.BlockSpec((S, 256), lambda b: (b, 0)),
            pl.BlockSpec((2, 256), lambda b: (0, 0)),
            pl.BlockSpec((1, 256), lambda b: (0, 0)),
            pl.BlockSpec((1, 256), lambda b: (0, 0)),
        ],
        out_specs=pl.BlockSpec((1, 256, S), lambda b: (b, 0, 0)),
        out_shape=jax.ShapeDtypeStruct((B, 256, S), jnp.float32),
    )(mx, mn, st2, g2row, be2row)


def kernel(xyz, points, W0, b0, g0, be0, W1, b1, g1, be1, W2, b2, g2, be2):
    w0x = W0[:, :3]
    w0p = W0[:, 3:]
    b0row = b0.reshape(1, 128)
    fps_out = _fps(xyz)
    new_xyz = fps_out[:, :3, :]
    s2row = fps_out[:, 3:4, :]
    new_xyz_t = jnp.transpose(new_xyz, (0, 2, 1))
    xyz_t = jnp.transpose(xyz, (0, 2, 1))
    idx, q, p = _bq(xyz_t, points, new_xyz_t, s2row, w0x, w0p, b0row, B)
    gidx = idx.reshape(M // _CHUNK, _CHUNK)
    g = _sc_gather_rows(p.reshape(B * N, 128), gidx, M)
    st0 = _stats0(g, q)
    y1, st1 = _passb(g, q, st0, g0.reshape(1, 128), be0.reshape(1, 128),
                     W1, b1.reshape(1, 128))
    mx, mn, st2 = _passc(y1, st1, g1.reshape(1, 128), be1.reshape(1, 128),
                         W2, b2.reshape(1, 256))
    new_points = _passd(mx, mn, st2, g2.reshape(1, 256), be2.reshape(1, 256))
    return (new_xyz, new_points)
